# baseline (speedup 1.0000x reference)
import functools

import jax
import jax.numpy as jnp
from jax import lax
from jax.experimental import pallas as pl
from jax.experimental.pallas import tpu as pltpu
from jax.experimental.pallas import tpu_sc as plsc

N = 10000
D = 128
E = 320000
G = 3

NC = 2
NS = 16
NW = NC * NS

NP = 10240
RPS = NP // NS

CB = 128
EPR = 2560
EP = EPR * CB
RW = EPR // NW

ZR = 64

R = 1024
NB = NP // R

_f32 = jnp.float32


CB2 = 32
NCH = RW * (CB // CB2)
ZCOPIES = RPS // CB2
PD = 4
RING = 2 * PD

_i32 = jnp.int32


def _unpack(packed_ref, c, slot_src, slot_dst):
    cpr = CB // CB2
    j = c // cpr
    h = (c % cpr) * CB2
    for k in range(CB2 // 16):
        p = packed_ref[j, pl.ds(h + k * 16, 16)]
        slot_src[pl.ds(k * 16, 16)] = jnp.bitwise_and(p, 0xFFFF)
        slot_dst[pl.ds(k * 16, 16)] = lax.shift_right_logical(p, 16)


def _sc_deg_body(pk_ref, degp_ref, dacc0, dacc1, dacc2, packed, dst_v, ones_v,
                 zb_v, sem):
    c = lax.axis_index("c")
    s = lax.axis_index("s")
    wid = c * NS + s

    def _fill_ones(i, _):
        ones_v[pl.ds(i * 16, 16)] = jnp.full((16,), 1.0, _f32)
        return 0

    def _fill_z(i, _):
        zb_v[pl.ds(i * 16, 16)] = jnp.zeros((16,), _f32)
        return 0

    lax.fori_loop(0, CB // 16, _fill_ones, 0)
    lax.fori_loop(0, RPS // 16, _fill_z, 0)

    for dacc in (dacc0, dacc1, dacc2):
        pltpu.sync_copy(zb_v, dacc.at[pl.ds(s * RPS, RPS)])
    plsc.subcore_barrier()

    base = wid * RW
    for g, dacc in enumerate((dacc0, dacc1, dacc2)):
        pltpu.sync_copy(pk_ref.at[g, pl.ds(base, RW)], packed)

        def _body(j, _, dacc=dacc):
            for k in range(CB // 16):
                p = packed[j, pl.ds(k * 16, 16)]
                dst_v[pl.ds(k * 16, 16)] = lax.shift_right_logical(p, 16)
            pltpu.sync_copy(ones_v, dacc.at[dst_v], add=True)
            return 0

        lax.fori_loop(0, RW, _body, 0)
    plsc.subcore_barrier()

    for g, dacc in enumerate((dacc0, dacc1, dacc2)):
        pltpu.sync_copy(dacc.at[pl.ds(s * RPS, RPS)],
                        degp_ref.at[pl.ds((c * G + g) * NP + s * RPS, RPS)])


def _sc_deg(packed_rows):
    fn = pl.kernel(
        _sc_deg_body,
        out_type=jax.ShapeDtypeStruct((NC * G * NP,), _f32),
        mesh=plsc.VectorSubcoreMesh(core_axis_name="c", subcore_axis_name="s",
                                    num_cores=NC, num_subcores=NS),
        scratch_types=[
            pltpu.VMEM_SHARED((NP,), _f32),
            pltpu.VMEM_SHARED((NP,), _f32),
            pltpu.VMEM_SHARED((NP,), _f32),
            pltpu.VMEM((RW, CB), _i32),
            pltpu.VMEM((CB,), _i32),
            pltpu.VMEM((CB,), _f32),
            pltpu.VMEM((RPS,), _f32),
            pltpu.SemaphoreType.DMA,
        ],
    )
    return fn(packed_rows)


def _sc_prop_body(v0, v1, v2, pk_ref, s0, s1, s2, acc, packed, *bufs):
    rows = bufs[0:RING]
    isrc = bufs[RING:2 * RING]
    idst = bufs[2 * RING:3 * RING]
    gsem = bufs[3 * RING:4 * RING]
    ssem = bufs[4 * RING:5 * RING]

    c_ax = lax.axis_index("c")
    s_ax = lax.axis_index("s")
    wid = c_ax * NS + s_ax
    base = wid * RW

    def g_start(b, vg):
        pltpu.async_copy(vg.at[isrc[b]], rows[b], gsem[b])

    def g_wait(b, vg):
        pltpu.make_async_copy(vg.at[isrc[b]], rows[b], gsem[b]).wait()

    def s_start(b):
        pltpu.async_copy(rows[b], acc.at[idst[b]], ssem[b], add=True)

    def s_wait(b):
        pltpu.make_async_copy(rows[b], acc.at[idst[b]], ssem[b]).wait()

    for g, (vg, sg) in enumerate(((v0, s0), (v1, s1), (v2, s2))):
        def _fz(i, _):
            rows[0][i // 8, pl.ds((i % 8) * 16, 16)] = jnp.zeros((16,), _f32)
            return 0

        lax.fori_loop(0, CB2 * 8, _fz, 0)
        for j in range(ZCOPIES):
            pltpu.sync_copy(rows[0], acc.at[pl.ds(s_ax * RPS + j * CB2, CB2)])
        plsc.subcore_barrier()

        pltpu.sync_copy(pk_ref.at[g, pl.ds(base, RW)], packed)

        for i in range(PD):
            _unpack(packed, i, isrc[i], idst[i])
            g_start(i, vg)
        for c in range(PD):
            b2 = (c + PD) % RING
            _unpack(packed, c + PD, isrc[b2], idst[b2])
            g_start(b2, vg)
            g_wait(c % RING, vg)
            s_start(c % RING)

        def _round(r, _, vg=vg):
            for b0 in range(RING):
                c = PD + r * RING + b0
                b = (PD + b0) % RING
                fb = b0
                s_wait(fb)
                _unpack(packed, c + PD, isrc[fb], idst[fb])
                g_start(fb, vg)
                g_wait(b, vg)
                s_start(b)
            return 0

        lax.fori_loop(0, (NCH - 2 * PD) // RING, _round, 0)

        for c in range(NCH - PD, NCH):
            s_wait((c - PD) % RING)
            g_wait(c % RING, vg)
            s_start(c % RING)
        for c in range(NCH - PD, NCH):
            s_wait(c % RING)

        plsc.subcore_barrier()
        pltpu.sync_copy(acc.at[pl.ds(s_ax * RPS, RPS)],
                        sg.at[pl.ds(c_ax * NP + s_ax * RPS, RPS)])


def _sc_prop(v0, v1, v2, packed_rows):
    fn = pl.kernel(
        _sc_prop_body,
        out_type=[jax.ShapeDtypeStruct((NC * NP, D), _f32)] * G,
        mesh=plsc.VectorSubcoreMesh(core_axis_name="c", subcore_axis_name="s",
                                    num_cores=NC, num_subcores=NS),
        scratch_types=(
            [pltpu.VMEM_SHARED((NP, D), _f32),
             pltpu.VMEM((RW, CB), _i32)]
            + [pltpu.VMEM((CB2, D), _f32)] * RING
            + [pltpu.VMEM((CB2,), _i32)] * (2 * RING)
            + [pltpu.SemaphoreType.DMA] * (2 * RING)
        ),
    )
    return fn(v0, v1, v2, packed_rows)


def _tc_mm_body(x_ref, wa_ref, wb_ref, wo_ref, bb_ref, bout_ref,
                y0_ref, y1_ref, y2_ref, w2_ref, cc_ref):
    for g, yref in enumerate((y0_ref, y1_ref, y2_ref)):
        yref[...] = jnp.dot(x_ref[...], wa_ref[g], preferred_element_type=_f32)
    cc = bout_ref[...][None, :]
    for g in range(G):
        w2_ref[g] = jnp.dot(wb_ref[g], wo_ref[g], preferred_element_type=_f32)
        cc = cc + jnp.dot(bb_ref[g][None, :], wo_ref[g],
                          preferred_element_type=_f32)
    cc_ref[...] = cc


def _tc_mm(x_pad, wa, wb, wo, bb, bout):
    full = lambda *shape: pl.BlockSpec(shape, lambda i: (0,) * len(shape))
    return pl.pallas_call(
        _tc_mm_body,
        grid=(NB,),
        in_specs=[
            pl.BlockSpec((R, D), lambda i: (i, 0)),
            full(G, D, D),
            full(G, D, D),
            full(G, D, D),
            full(G, D),
            full(D),
        ],
        out_specs=[
            pl.BlockSpec((R, D), lambda i: (i, 0)),
            pl.BlockSpec((R, D), lambda i: (i, 0)),
            pl.BlockSpec((R, D), lambda i: (i, 0)),
            full(G, D, D),
            full(1, D),
        ],
        out_shape=[
            jax.ShapeDtypeStruct((NP, D), _f32),
            jax.ShapeDtypeStruct((NP, D), _f32),
            jax.ShapeDtypeStruct((NP, D), _f32),
            jax.ShapeDtypeStruct((G, D, D), _f32),
            jax.ShapeDtypeStruct((1, D), _f32),
        ],
    )(x_pad, wa, wb, wo, bb, bout)


def _tc_scale_body(y0_ref, y1_ref, y2_ref, degp_ref, v0_ref, v1_ref, v2_ref,
                   dinv_ref):
    deg = degp_ref[0] + degp_ref[1] + 1.0
    di = lax.rsqrt(deg)
    dinv_ref[...] = di
    for g, (yref, vref) in enumerate(((y0_ref, v0_ref), (y1_ref, v1_ref),
                                      (y2_ref, v2_ref))):
        vref[...] = di[g][:, None] * yref[...]


def _tc_scale(y0, y1, y2, degp):
    vspec = pl.BlockSpec((R, D), lambda i: (i, 0))
    return pl.pallas_call(
        _tc_scale_body,
        grid=(NB,),
        in_specs=[vspec, vspec, vspec,
                  pl.BlockSpec((NC, G, R), lambda i: (0, 0, i))],
        out_specs=[vspec, vspec, vspec,
                   pl.BlockSpec((G, R), lambda i: (0, i))],
        out_shape=[
            jax.ShapeDtypeStruct((NP, D), _f32),
            jax.ShapeDtypeStruct((NP, D), _f32),
            jax.ShapeDtypeStruct((NP, D), _f32),
            jax.ShapeDtypeStruct((G, NP), _f32),
        ],
    )(y0, y1, y2, degp)


def _tc_mid_body(s0_ref, s1_ref, s2_ref, v0_ref, v1_ref, v2_ref, dinv_ref,
                 ba_ref, w2_ref, w0_ref, w1_ref, w2o_ref):
    di = dinv_ref[...]
    srefs = (s0_ref, s1_ref, s2_ref)
    vrefs = (v0_ref, v1_ref, v2_ref)
    wrefs = (w0_ref, w1_ref, w2o_ref)
    for g in range(G):
        u = di[g][:, None] * (srefs[g][0] + srefs[g][1] + vrefs[g][...])
        h1 = jnp.maximum(u + ba_ref[g][None, :], 0.0)
        wrefs[g][...] = di[g][:, None] * jnp.dot(
            h1, w2_ref[g], preferred_element_type=_f32)


def _tc_mid(s0, s1, s2, v0, v1, v2, dinv, ba, w2):
    full = lambda *shape: pl.BlockSpec(shape, lambda i: (0,) * len(shape))
    part = pl.BlockSpec((NC, R, D), lambda i: (0, i, 0))
    vspec = pl.BlockSpec((R, D), lambda i: (i, 0))
    return pl.pallas_call(
        _tc_mid_body,
        grid=(NB,),
        in_specs=[part, part, part, vspec, vspec, vspec,
                  pl.BlockSpec((G, R), lambda i: (0, i)),
                  full(G, D), full(G, D, D)],
        out_specs=[vspec, vspec, vspec],
        out_shape=[jax.ShapeDtypeStruct((NP, D), _f32)] * G,
    )(s0, s1, s2, v0, v1, v2, dinv, ba, w2)


def _tc_final_body(t0_ref, t1_ref, t2_ref, w0_ref, w1_ref, w2_ref, dinv_ref,
                   cc_ref, out_ref):
    di = dinv_ref[...]
    trefs = (t0_ref, t1_ref, t2_ref)
    wrefs = (w0_ref, w1_ref, w2_ref)
    o = jnp.broadcast_to(cc_ref[...], (R, D))
    for g in range(G):
        o = o + di[g][:, None] * (trefs[g][0] + trefs[g][1] + wrefs[g][...])
    out_ref[...] = o


def _tc_final(t0, t1, t2, w0, w1, w2, dinv, cc):
    full = lambda *shape: pl.BlockSpec(shape, lambda i: (0,) * len(shape))
    part = pl.BlockSpec((NC, R, D), lambda i: (0, i, 0))
    vspec = pl.BlockSpec((R, D), lambda i: (i, 0))
    return pl.pallas_call(
        _tc_final_body,
        grid=(NB,),
        in_specs=[part, part, part, vspec, vspec, vspec,
                  pl.BlockSpec((G, R), lambda i: (0, i)),
                  full(1, D)],
        out_specs=vspec,
        out_shape=jax.ShapeDtypeStruct((NP, D), _f32),
    )(t0, t1, t2, w0, w1, w2, dinv, cc)


def kernel(x, edge_indices, W0_0, b0_0, W0_1, b0_1, W1_0, b1_0, W1_1, b1_1,
           W2_0, b2_0, W2_1, b2_1, Wout, bout):
    x_pad = jnp.pad(x, ((0, NP - N), (0, 0)))
    packed_real = jnp.bitwise_or(edge_indices[:, 0, :],
                                 edge_indices[:, 1, :] << 16)
    pad_node = N + jnp.arange(EP - E, dtype=jnp.int32) % (NP - N)
    pad_packed = jnp.broadcast_to(pad_node | (pad_node << 16), (G, EP - E))
    pk_rows = jnp.concatenate([packed_real, pad_packed], axis=1
                              ).reshape(G, EPR, CB)
    perm = (jnp.arange(EPR, dtype=jnp.int32).reshape(RW, NW).T).reshape(-1)
    pk_rows = pk_rows[:, perm]

    wa = jnp.stack([W0_0, W1_0, W2_0])
    ba = jnp.stack([b0_0, b1_0, b2_0])
    wb = jnp.stack([W0_1, W1_1, W2_1])
    bb = jnp.stack([b0_1, b1_1, b2_1])
    wo = Wout.reshape(G, D, D)

    y0, y1, y2, w2, cc = _tc_mm(x_pad, wa, wb, wo, bb, bout)
    degp = _sc_deg(pk_rows).reshape(NC, G, NP)
    v0, v1, v2, dinv = _tc_scale(y0, y1, y2, degp)
    s0, s1, s2 = (a.reshape(NC, NP, D) for a in _sc_prop(v0, v1, v2, pk_rows))
    w0, w1, w2o = _tc_mid(s0, s1, s2, v0, v1, v2, dinv, ba, w2)
    t0, t1, t2 = (a.reshape(NC, NP, D) for a in _sc_prop(w0, w1, w2o, pk_rows))
    out = _tc_final(t0, t1, t2, w0, w1, w2o, dinv, cc)
    return out[:N]


# speedup vs baseline: 1.0137x; 1.0137x over previous
import functools

import jax
import jax.numpy as jnp
from jax import lax
from jax.experimental import pallas as pl
from jax.experimental.pallas import tpu as pltpu
from jax.experimental.pallas import tpu_sc as plsc

N = 10000
D = 128
E = 320000
G = 3

NC = 2
NS = 16
NW = NC * NS

NP = 10240
RPS = NP // NS

CB = 128
EPR = 2560
EP = EPR * CB
RW = EPR // NW

ZR = 64

R = 1024
NB = NP // R

_f32 = jnp.float32


CB2 = 32
NCH = RW * (CB // CB2)
ZCOPIES = RPS // CB2
PD = 4
RING = 2 * PD

_i32 = jnp.int32


def _unpack(packed_ref, c, slot_src, slot_dst):
    cpr = CB // CB2
    j = c // cpr
    h = (c % cpr) * CB2
    for k in range(CB2 // 16):
        p = packed_ref[j, pl.ds(h + k * 16, 16)]
        slot_src[pl.ds(k * 16, 16)] = jnp.bitwise_and(p, 0xFFFF)
        slot_dst[pl.ds(k * 16, 16)] = lax.shift_right_logical(p, 16)


def _sc_deg_body(pk_ref, degp_ref, dacc0, dacc1, dacc2, packed, dst_v, ones_v,
                 zb_v, sem):
    c = lax.axis_index("c")
    s = lax.axis_index("s")
    wid = c * NS + s

    def _fill_ones(i, _):
        ones_v[pl.ds(i * 16, 16)] = jnp.full((16,), 1.0, _f32)
        return 0

    def _fill_z(i, _):
        zb_v[pl.ds(i * 16, 16)] = jnp.zeros((16,), _f32)
        return 0

    lax.fori_loop(0, CB // 16, _fill_ones, 0)
    lax.fori_loop(0, RPS // 16, _fill_z, 0)

    for dacc in (dacc0, dacc1, dacc2):
        pltpu.sync_copy(zb_v, dacc.at[pl.ds(s * RPS, RPS)])
    plsc.subcore_barrier()

    base = wid * RW
    for g, dacc in enumerate((dacc0, dacc1, dacc2)):
        pltpu.sync_copy(pk_ref.at[g, pl.ds(base, RW)], packed)

        def _body(j, _, dacc=dacc):
            for k in range(CB // 16):
                p = packed[j, pl.ds(k * 16, 16)]
                dst_v[pl.ds(k * 16, 16)] = lax.shift_right_logical(p, 16)
            pltpu.sync_copy(ones_v, dacc.at[dst_v], add=True)
            return 0

        lax.fori_loop(0, RW, _body, 0)
    plsc.subcore_barrier()

    for g, dacc in enumerate((dacc0, dacc1, dacc2)):
        pltpu.sync_copy(dacc.at[pl.ds(s * RPS, RPS)],
                        degp_ref.at[pl.ds((c * G + g) * NP + s * RPS, RPS)])


def _sc_deg(packed_rows):
    fn = pl.kernel(
        _sc_deg_body,
        out_type=jax.ShapeDtypeStruct((NC * G * NP,), _f32),
        mesh=plsc.VectorSubcoreMesh(core_axis_name="c", subcore_axis_name="s",
                                    num_cores=NC, num_subcores=NS),
        scratch_types=[
            pltpu.VMEM_SHARED((NP,), _f32),
            pltpu.VMEM_SHARED((NP,), _f32),
            pltpu.VMEM_SHARED((NP,), _f32),
            pltpu.VMEM((RW, CB), _i32),
            pltpu.VMEM((CB,), _i32),
            pltpu.VMEM((CB,), _f32),
            pltpu.VMEM((RPS,), _f32),
            pltpu.SemaphoreType.DMA,
        ],
    )
    return fn(packed_rows)


def _sc_prop_body(v0, v1, v2, pk_ref, s0, s1, s2, acc, packed, *bufs,
                  dt=jnp.float32):
    lanes = 16 if dt == jnp.float32 else 32
    grp = D // lanes
    rows = bufs[0:RING]
    isrc = bufs[RING:2 * RING]
    idst = bufs[2 * RING:3 * RING]
    gsem = bufs[3 * RING:4 * RING]
    ssem = bufs[4 * RING:5 * RING]

    c_ax = lax.axis_index("c")
    s_ax = lax.axis_index("s")
    wid = c_ax * NS + s_ax
    base = wid * RW

    def g_start(b, vg):
        pltpu.async_copy(vg.at[isrc[b]], rows[b], gsem[b])

    def g_wait(b, vg):
        pltpu.make_async_copy(vg.at[isrc[b]], rows[b], gsem[b]).wait()

    def s_start(b):
        pltpu.async_copy(rows[b], acc.at[idst[b]], ssem[b], add=True)

    def s_wait(b):
        pltpu.make_async_copy(rows[b], acc.at[idst[b]], ssem[b]).wait()

    for g, (vg, sg) in enumerate(((v0, s0), (v1, s1), (v2, s2))):
        if dt == jnp.float32:
            def _fz(i, _):
                rows[0][i // grp, pl.ds((i % grp) * lanes, lanes)] = \
                    jnp.zeros((lanes,), dt)
                return 0

            lax.fori_loop(0, CB2 * grp, _fz, 0)
        else:
            for r2 in range(CB2 // 2):
                for k in range(D // 16):
                    rows[0][pl.ds(2 * r2, 2), pl.ds(k * 16, 16)] = \
                        jnp.zeros((2, 16), dt)
        for j in range(ZCOPIES):
            pltpu.sync_copy(rows[0], acc.at[pl.ds(s_ax * RPS + j * CB2, CB2)])
        plsc.subcore_barrier()

        pltpu.sync_copy(pk_ref.at[g, pl.ds(base, RW)], packed)

        for i in range(PD):
            _unpack(packed, i, isrc[i], idst[i])
            g_start(i, vg)
        for c in range(PD):
            b2 = (c + PD) % RING
            _unpack(packed, c + PD, isrc[b2], idst[b2])
            g_start(b2, vg)
            g_wait(c % RING, vg)
            s_start(c % RING)

        def _round(r, _, vg=vg):
            for b0 in range(RING):
                c = PD + r * RING + b0
                b = (PD + b0) % RING
                fb = b0
                s_wait(fb)
                _unpack(packed, c + PD, isrc[fb], idst[fb])
                g_start(fb, vg)
                g_wait(b, vg)
                s_start(b)
            return 0

        lax.fori_loop(0, (NCH - 2 * PD) // RING, _round, 0)

        for c in range(NCH - PD, NCH):
            s_wait((c - PD) % RING)
            g_wait(c % RING, vg)
            s_start(c % RING)
        for c in range(NCH - PD, NCH):
            s_wait(c % RING)

        plsc.subcore_barrier()
        pltpu.sync_copy(acc.at[pl.ds(s_ax * RPS, RPS)],
                        sg.at[pl.ds(c_ax * NP + s_ax * RPS, RPS)])


def _sc_prop(v0, v1, v2, packed_rows, dt=jnp.float32):
    fn = pl.kernel(
        functools.partial(_sc_prop_body, dt=dt),
        out_type=[jax.ShapeDtypeStruct((NC * NP, D), dt)] * G,
        mesh=plsc.VectorSubcoreMesh(core_axis_name="c", subcore_axis_name="s",
                                    num_cores=NC, num_subcores=NS),
        scratch_types=(
            [pltpu.VMEM_SHARED((NP, D), dt),
             pltpu.VMEM((RW, CB), _i32)]
            + [pltpu.VMEM((CB2, D), dt)] * RING
            + [pltpu.VMEM((CB2,), _i32)] * (2 * RING)
            + [pltpu.SemaphoreType.DMA] * (2 * RING)
        ),
    )
    return fn(v0, v1, v2, packed_rows)


def _tc_mm_body(x_ref, wa_ref, wb_ref, wo_ref, bb_ref, bout_ref,
                y0_ref, y1_ref, y2_ref, w2_ref, cc_ref):
    for g, yref in enumerate((y0_ref, y1_ref, y2_ref)):
        yref[...] = jnp.dot(x_ref[...], wa_ref[g], preferred_element_type=_f32)
    cc = bout_ref[...][None, :]
    for g in range(G):
        w2_ref[g] = jnp.dot(wb_ref[g], wo_ref[g], preferred_element_type=_f32)
        cc = cc + jnp.dot(bb_ref[g][None, :], wo_ref[g],
                          preferred_element_type=_f32)
    cc_ref[...] = cc


def _tc_mm(x_pad, wa, wb, wo, bb, bout):
    full = lambda *shape: pl.BlockSpec(shape, lambda i: (0,) * len(shape))
    return pl.pallas_call(
        _tc_mm_body,
        grid=(NB,),
        in_specs=[
            pl.BlockSpec((R, D), lambda i: (i, 0)),
            full(G, D, D),
            full(G, D, D),
            full(G, D, D),
            full(G, D),
            full(D),
        ],
        out_specs=[
            pl.BlockSpec((R, D), lambda i: (i, 0)),
            pl.BlockSpec((R, D), lambda i: (i, 0)),
            pl.BlockSpec((R, D), lambda i: (i, 0)),
            full(G, D, D),
            full(1, D),
        ],
        out_shape=[
            jax.ShapeDtypeStruct((NP, D), _f32),
            jax.ShapeDtypeStruct((NP, D), _f32),
            jax.ShapeDtypeStruct((NP, D), _f32),
            jax.ShapeDtypeStruct((G, D, D), _f32),
            jax.ShapeDtypeStruct((1, D), _f32),
        ],
    )(x_pad, wa, wb, wo, bb, bout)


def _tc_scale_body(y0_ref, y1_ref, y2_ref, degp_ref, v0_ref, v1_ref, v2_ref,
                   dinv_ref):
    deg = degp_ref[0] + degp_ref[1] + 1.0
    di = lax.rsqrt(deg)
    dinv_ref[...] = di
    for g, (yref, vref) in enumerate(((y0_ref, v0_ref), (y1_ref, v1_ref),
                                      (y2_ref, v2_ref))):
        vref[...] = di[g][:, None] * yref[...]


def _tc_scale(y0, y1, y2, degp):
    vspec = pl.BlockSpec((R, D), lambda i: (i, 0))
    return pl.pallas_call(
        _tc_scale_body,
        grid=(NB,),
        in_specs=[vspec, vspec, vspec,
                  pl.BlockSpec((NC, G, R), lambda i: (0, 0, i))],
        out_specs=[vspec] * 3 + [pl.BlockSpec((G, R), lambda i: (0, i))],
        out_shape=(
            [jax.ShapeDtypeStruct((NP, D), _f32)] * 3
            + [jax.ShapeDtypeStruct((G, NP), _f32)]
        ),
    )(y0, y1, y2, degp)


def _tc_mid_body(s0_ref, s1_ref, s2_ref, v0_ref, v1_ref, v2_ref, dinv_ref,
                 ba_ref, w2_ref, w0_ref, w1_ref, w2o_ref):
    di = dinv_ref[...]
    srefs = (s0_ref, s1_ref, s2_ref)
    vrefs = (v0_ref, v1_ref, v2_ref)
    wrefs = (w0_ref, w1_ref, w2o_ref)
    for g in range(G):
        ssum = (srefs[g][0].astype(_f32) + srefs[g][1].astype(_f32))
        u = di[g][:, None] * (ssum + vrefs[g][...])
        h1 = jnp.maximum(u + ba_ref[g][None, :], 0.0)
        wrefs[g][...] = di[g][:, None] * jnp.dot(
            h1, w2_ref[g], preferred_element_type=_f32)


def _tc_mid(s0, s1, s2, v0, v1, v2, dinv, ba, w2):
    full = lambda *shape: pl.BlockSpec(shape, lambda i: (0,) * len(shape))
    part = pl.BlockSpec((NC, R, D), lambda i: (0, i, 0))
    vspec = pl.BlockSpec((R, D), lambda i: (i, 0))
    return pl.pallas_call(
        _tc_mid_body,
        grid=(NB,),
        in_specs=[part, part, part, vspec, vspec, vspec,
                  pl.BlockSpec((G, R), lambda i: (0, i)),
                  full(G, D), full(G, D, D)],
        out_specs=[vspec, vspec, vspec],
        out_shape=[jax.ShapeDtypeStruct((NP, D), _f32)] * G,
    )(s0, s1, s2, v0, v1, v2, dinv, ba, w2)


def _tc_final_body(t0_ref, t1_ref, t2_ref, w0_ref, w1_ref, w2_ref, dinv_ref,
                   cc_ref, out_ref):
    di = dinv_ref[...]
    trefs = (t0_ref, t1_ref, t2_ref)
    wrefs = (w0_ref, w1_ref, w2_ref)
    o = jnp.broadcast_to(cc_ref[...], (R, D))
    for g in range(G):
        o = o + di[g][:, None] * (trefs[g][0] + trefs[g][1] + wrefs[g][...])
    out_ref[...] = o


def _tc_final(t0, t1, t2, w0, w1, w2, dinv, cc):
    full = lambda *shape: pl.BlockSpec(shape, lambda i: (0,) * len(shape))
    part = pl.BlockSpec((NC, R, D), lambda i: (0, i, 0))
    vspec = pl.BlockSpec((R, D), lambda i: (i, 0))
    return pl.pallas_call(
        _tc_final_body,
        grid=(NB,),
        in_specs=[part, part, part, vspec, vspec, vspec,
                  pl.BlockSpec((G, R), lambda i: (0, i)),
                  full(1, D)],
        out_specs=vspec,
        out_shape=jax.ShapeDtypeStruct((NP, D), _f32),
    )(t0, t1, t2, w0, w1, w2, dinv, cc)


def kernel(x, edge_indices, W0_0, b0_0, W0_1, b0_1, W1_0, b1_0, W1_1, b1_1,
           W2_0, b2_0, W2_1, b2_1, Wout, bout):
    x_pad = jnp.pad(x, ((0, NP - N), (0, 0)))
    packed_real = jnp.bitwise_or(edge_indices[:, 0, :],
                                 edge_indices[:, 1, :] << 16)
    pad_node = N + jnp.arange(EP - E, dtype=jnp.int32) % (NP - N)
    pad_packed = jnp.broadcast_to(pad_node | (pad_node << 16), (G, EP - E))
    pk_rows = (jnp.concatenate([packed_real, pad_packed], axis=1)
               .reshape(G, RW, NW, CB).transpose(0, 2, 1, 3)
               .reshape(G, EPR, CB))

    wa = jnp.stack([W0_0, W1_0, W2_0])
    ba = jnp.stack([b0_0, b1_0, b2_0])
    wb = jnp.stack([W0_1, W1_1, W2_1])
    bb = jnp.stack([b0_1, b1_1, b2_1])
    wo = Wout.reshape(G, D, D)

    degp = _sc_deg(pk_rows).reshape(NC, G, NP)
    y0, y1, y2, w2, cc = _tc_mm(x_pad, wa, wb, wo, bb, bout)
    v0, v1, v2, dinv = _tc_scale(y0, y1, y2, degp)
    s0, s1, s2 = (a.reshape(NC, NP, D) for a in _sc_prop(v0, v1, v2, pk_rows))
    w0, w1, w2o = _tc_mid(s0, s1, s2, v0, v1, v2, dinv, ba, w2)
    t0, t1, t2 = (a.reshape(NC, NP, D) for a in _sc_prop(w0, w1, w2o, pk_rows))
    out = _tc_final(t0, t1, t2, w0, w1, w2o, dinv, cc)
    return out[:N]


# speedup vs baseline: 1.0331x; 1.0191x over previous
import functools

import jax
import jax.numpy as jnp
from jax import lax
from jax.experimental import pallas as pl
from jax.experimental.pallas import tpu as pltpu
from jax.experimental.pallas import tpu_sc as plsc

N = 10000
D = 128
E = 320000
G = 3

NC = 2
NS = 16
NW = NC * NS

NP = 10240
RPS = NP // NS

CB = 128
EPR = 2560
EP = EPR * CB
RW = EPR // NW

ZR = 64

R = 1024
NB = NP // R

_f32 = jnp.float32


CB2 = 32
NCH = RW * (CB // CB2)
ZCOPIES = RPS // CB2
PD = 4
RING = 2 * PD

_i32 = jnp.int32


def _unpack(packed_ref, c, slot_src, slot_dst):
    cpr = CB // CB2
    j = c // cpr
    h = (c % cpr) * CB2
    for k in range(CB2 // 16):
        p = packed_ref[j, pl.ds(h + k * 16, 16)]
        slot_src[pl.ds(k * 16, 16)] = jnp.bitwise_and(p, 0xFFFF)
        slot_dst[pl.ds(k * 16, 16)] = lax.shift_right_logical(p, 16)


def _sc_deg_body(pk_ref, degp_ref, dacc0, dacc1, dacc2, packed, dst0, dst1,
                 ones_v, zb_v, sm0, sm1):
    c = lax.axis_index("c")
    s = lax.axis_index("s")
    wid = c * NS + s
    dsts = (dst0, dst1)
    sems = (sm0, sm1)

    def _fill_ones(i, _):
        ones_v[pl.ds(i * 16, 16)] = jnp.full((16,), 1.0, _f32)
        return 0

    def _fill_z(i, _):
        zb_v[pl.ds(i * 16, 16)] = jnp.zeros((16,), _f32)
        return 0

    lax.fori_loop(0, CB // 16, _fill_ones, 0)
    lax.fori_loop(0, RPS // 16, _fill_z, 0)

    for dacc in (dacc0, dacc1, dacc2):
        pltpu.sync_copy(zb_v, dacc.at[pl.ds(s * RPS, RPS)])
    plsc.subcore_barrier()

    base = wid * RW
    for g, dacc in enumerate((dacc0, dacc1, dacc2)):
        pltpu.sync_copy(pk_ref.at[g, pl.ds(base, RW)], packed)

        def _unp(j, b):
            for k in range(CB // 16):
                p = packed[j, pl.ds(k * 16, 16)]
                dsts[b][pl.ds(k * 16, 16)] = lax.shift_right_logical(p, 16)

        def s_start(b, dacc=dacc):
            pltpu.async_copy(ones_v, dacc.at[dsts[b]], sems[b], add=True)

        def s_wait(b, dacc=dacc):
            pltpu.make_async_copy(ones_v, dacc.at[dsts[b]], sems[b]).wait()

        _unp(0, 0)
        s_start(0)
        _unp(1, 1)
        s_start(1)

        def _body(r, _):
            for b in range(2):
                s_wait(b)
                _unp(2 + r * 2 + b, b)
                s_start(b)
            return 0

        lax.fori_loop(0, (RW - 2) // 2, _body, 0)
        s_wait(0)
        s_wait(1)
    plsc.subcore_barrier()

    for g, dacc in enumerate((dacc0, dacc1, dacc2)):
        pltpu.sync_copy(dacc.at[pl.ds(s * RPS, RPS)],
                        degp_ref.at[pl.ds((c * G + g) * NP + s * RPS, RPS)])


def _sc_deg(packed_rows):
    fn = pl.kernel(
        _sc_deg_body,
        out_type=jax.ShapeDtypeStruct((NC * G * NP,), _f32),
        mesh=plsc.VectorSubcoreMesh(core_axis_name="c", subcore_axis_name="s",
                                    num_cores=NC, num_subcores=NS),
        scratch_types=[
            pltpu.VMEM_SHARED((NP,), _f32),
            pltpu.VMEM_SHARED((NP,), _f32),
            pltpu.VMEM_SHARED((NP,), _f32),
            pltpu.VMEM((RW, CB), _i32),
            pltpu.VMEM((CB,), _i32),
            pltpu.VMEM((CB,), _i32),
            pltpu.VMEM((CB,), _f32),
            pltpu.VMEM((RPS,), _f32),
            pltpu.SemaphoreType.DMA,
            pltpu.SemaphoreType.DMA,
        ],
    )
    return fn(packed_rows)


def _sc_prop_body(v0, v1, v2, pk_ref, s0, s1, s2, acc, packed, *bufs,
                  dt=jnp.float32):
    lanes = 16 if dt == jnp.float32 else 32
    grp = D // lanes
    rows = bufs[0:RING]
    isrc = bufs[RING:2 * RING]
    idst = bufs[2 * RING:3 * RING]
    gsem = bufs[3 * RING:4 * RING]
    ssem = bufs[4 * RING:5 * RING]

    c_ax = lax.axis_index("c")
    s_ax = lax.axis_index("s")
    wid = c_ax * NS + s_ax
    base = wid * RW

    def g_start(b, vg):
        pltpu.async_copy(vg.at[isrc[b]], rows[b], gsem[b])

    def g_wait(b, vg):
        pltpu.make_async_copy(vg.at[isrc[b]], rows[b], gsem[b]).wait()

    def s_start(b):
        pltpu.async_copy(rows[b], acc.at[idst[b]], ssem[b], add=True)

    def s_wait(b):
        pltpu.make_async_copy(rows[b], acc.at[idst[b]], ssem[b]).wait()

    for g, (vg, sg) in enumerate(((v0, s0), (v1, s1), (v2, s2))):
        if dt == jnp.float32:
            def _fz(i, _):
                rows[0][i // grp, pl.ds((i % grp) * lanes, lanes)] = \
                    jnp.zeros((lanes,), dt)
                return 0

            lax.fori_loop(0, CB2 * grp, _fz, 0)
        else:
            for r2 in range(CB2 // 2):
                for k in range(D // 16):
                    rows[0][pl.ds(2 * r2, 2), pl.ds(k * 16, 16)] = \
                        jnp.zeros((2, 16), dt)
        for j in range(ZCOPIES):
            pltpu.sync_copy(rows[0], acc.at[pl.ds(s_ax * RPS + j * CB2, CB2)])
        plsc.subcore_barrier()

        pltpu.sync_copy(pk_ref.at[g, pl.ds(base, RW)], packed)

        for i in range(PD):
            _unpack(packed, i, isrc[i], idst[i])
            g_start(i, vg)
        for c in range(PD):
            b2 = (c + PD) % RING
            _unpack(packed, c + PD, isrc[b2], idst[b2])
            g_start(b2, vg)
            g_wait(c % RING, vg)
            s_start(c % RING)

        def _round(r, _, vg=vg):
            for b0 in range(RING):
                c = PD + r * RING + b0
                b = (PD + b0) % RING
                fb = b0
                s_wait(fb)
                _unpack(packed, c + PD, isrc[fb], idst[fb])
                g_start(fb, vg)
                g_wait(b, vg)
                s_start(b)
            return 0

        lax.fori_loop(0, (NCH - 2 * PD) // RING, _round, 0)

        for c in range(NCH - PD, NCH):
            s_wait((c - PD) % RING)
            g_wait(c % RING, vg)
            s_start(c % RING)
        for c in range(NCH - PD, NCH):
            s_wait(c % RING)

        plsc.subcore_barrier()
        pltpu.sync_copy(acc.at[pl.ds(s_ax * RPS, RPS)],
                        sg.at[pl.ds(c_ax * NP + s_ax * RPS, RPS)])


def _sc_prop(v0, v1, v2, packed_rows, dt=jnp.float32):
    fn = pl.kernel(
        functools.partial(_sc_prop_body, dt=dt),
        out_type=[jax.ShapeDtypeStruct((NC * NP, D), dt)] * G,
        mesh=plsc.VectorSubcoreMesh(core_axis_name="c", subcore_axis_name="s",
                                    num_cores=NC, num_subcores=NS),
        scratch_types=(
            [pltpu.VMEM_SHARED((NP, D), dt),
             pltpu.VMEM((RW, CB), _i32)]
            + [pltpu.VMEM((CB2, D), dt)] * RING
            + [pltpu.VMEM((CB2,), _i32)] * (2 * RING)
            + [pltpu.SemaphoreType.DMA] * (2 * RING)
        ),
    )
    return fn(v0, v1, v2, packed_rows)


def _tc_mm_body(x_ref, wa_ref, wb_ref, wo_ref, bb_ref, bout_ref,
                y0_ref, y1_ref, y2_ref, w2_ref, cc_ref):
    for g, yref in enumerate((y0_ref, y1_ref, y2_ref)):
        yref[...] = jnp.dot(x_ref[...], wa_ref[g], preferred_element_type=_f32)
    cc = bout_ref[...][None, :]
    for g in range(G):
        w2_ref[g] = jnp.dot(wb_ref[g], wo_ref[g], preferred_element_type=_f32)
        cc = cc + jnp.dot(bb_ref[g][None, :], wo_ref[g],
                          preferred_element_type=_f32)
    cc_ref[...] = cc


def _tc_mm(x_pad, wa, wb, wo, bb, bout):
    full = lambda *shape: pl.BlockSpec(shape, lambda i: (0,) * len(shape))
    return pl.pallas_call(
        _tc_mm_body,
        grid=(NB,),
        in_specs=[
            pl.BlockSpec((R, D), lambda i: (i, 0)),
            full(G, D, D),
            full(G, D, D),
            full(G, D, D),
            full(G, D),
            full(D),
        ],
        out_specs=[
            pl.BlockSpec((R, D), lambda i: (i, 0)),
            pl.BlockSpec((R, D), lambda i: (i, 0)),
            pl.BlockSpec((R, D), lambda i: (i, 0)),
            full(G, D, D),
            full(1, D),
        ],
        out_shape=[
            jax.ShapeDtypeStruct((NP, D), _f32),
            jax.ShapeDtypeStruct((NP, D), _f32),
            jax.ShapeDtypeStruct((NP, D), _f32),
            jax.ShapeDtypeStruct((G, D, D), _f32),
            jax.ShapeDtypeStruct((1, D), _f32),
        ],
    )(x_pad, wa, wb, wo, bb, bout)


def _tc_scale_body(y0_ref, y1_ref, y2_ref, degp_ref, v0_ref, v1_ref, v2_ref,
                   dinv_ref):
    deg = degp_ref[0] + degp_ref[1] + 1.0
    di = lax.rsqrt(deg)
    dinv_ref[...] = di
    for g, (yref, vref) in enumerate(((y0_ref, v0_ref), (y1_ref, v1_ref),
                                      (y2_ref, v2_ref))):
        vref[...] = di[g][:, None] * yref[...]


def _tc_scale(y0, y1, y2, degp):
    vspec = pl.BlockSpec((R, D), lambda i: (i, 0))
    return pl.pallas_call(
        _tc_scale_body,
        grid=(NB,),
        in_specs=[vspec, vspec, vspec,
                  pl.BlockSpec((NC, G, R), lambda i: (0, 0, i))],
        out_specs=[vspec] * 3 + [pl.BlockSpec((G, R), lambda i: (0, i))],
        out_shape=(
            [jax.ShapeDtypeStruct((NP, D), _f32)] * 3
            + [jax.ShapeDtypeStruct((G, NP), _f32)]
        ),
    )(y0, y1, y2, degp)


def _tc_mid_body(s0_ref, s1_ref, s2_ref, v0_ref, v1_ref, v2_ref, dinv_ref,
                 ba_ref, w2_ref, w0_ref, w1_ref, w2o_ref):
    di = dinv_ref[...]
    srefs = (s0_ref, s1_ref, s2_ref)
    vrefs = (v0_ref, v1_ref, v2_ref)
    wrefs = (w0_ref, w1_ref, w2o_ref)
    for g in range(G):
        ssum = (srefs[g][0].astype(_f32) + srefs[g][1].astype(_f32))
        u = di[g][:, None] * (ssum + vrefs[g][...])
        h1 = jnp.maximum(u + ba_ref[g][None, :], 0.0)
        wrefs[g][...] = di[g][:, None] * jnp.dot(
            h1, w2_ref[g], preferred_element_type=_f32)


def _tc_mid(s0, s1, s2, v0, v1, v2, dinv, ba, w2):
    full = lambda *shape: pl.BlockSpec(shape, lambda i: (0,) * len(shape))
    part = pl.BlockSpec((NC, R, D), lambda i: (0, i, 0))
    vspec = pl.BlockSpec((R, D), lambda i: (i, 0))
    return pl.pallas_call(
        _tc_mid_body,
        grid=(NB,),
        in_specs=[part, part, part, vspec, vspec, vspec,
                  pl.BlockSpec((G, R), lambda i: (0, i)),
                  full(G, D), full(G, D, D)],
        out_specs=[vspec, vspec, vspec],
        out_shape=[jax.ShapeDtypeStruct((NP, D), _f32)] * G,
    )(s0, s1, s2, v0, v1, v2, dinv, ba, w2)


def _tc_final_body(t0_ref, t1_ref, t2_ref, w0_ref, w1_ref, w2_ref, dinv_ref,
                   cc_ref, out_ref):
    di = dinv_ref[...]
    trefs = (t0_ref, t1_ref, t2_ref)
    wrefs = (w0_ref, w1_ref, w2_ref)
    o = jnp.broadcast_to(cc_ref[...], (R, D))
    for g in range(G):
        o = o + di[g][:, None] * (trefs[g][0] + trefs[g][1] + wrefs[g][...])
    out_ref[...] = o


def _tc_final(t0, t1, t2, w0, w1, w2, dinv, cc):
    full = lambda *shape: pl.BlockSpec(shape, lambda i: (0,) * len(shape))
    part = pl.BlockSpec((NC, R, D), lambda i: (0, i, 0))
    vspec = pl.BlockSpec((R, D), lambda i: (i, 0))
    return pl.pallas_call(
        _tc_final_body,
        grid=(NB,),
        in_specs=[part, part, part, vspec, vspec, vspec,
                  pl.BlockSpec((G, R), lambda i: (0, i)),
                  full(1, D)],
        out_specs=vspec,
        out_shape=jax.ShapeDtypeStruct((NP, D), _f32),
    )(t0, t1, t2, w0, w1, w2, dinv, cc)


def kernel(x, edge_indices, W0_0, b0_0, W0_1, b0_1, W1_0, b1_0, W1_1, b1_1,
           W2_0, b2_0, W2_1, b2_1, Wout, bout):
    x_pad = jnp.pad(x, ((0, NP - N), (0, 0)))
    packed_real = jnp.bitwise_or(edge_indices[:, 0, :],
                                 edge_indices[:, 1, :] << 16)
    pad_node = N + jnp.arange(EP - E, dtype=jnp.int32) % (NP - N)
    pad_packed = jnp.broadcast_to(pad_node | (pad_node << 16), (G, EP - E))
    pk_rows = (jnp.concatenate([packed_real, pad_packed], axis=1)
               .reshape(G, RW, NW, CB).transpose(0, 2, 1, 3)
               .reshape(G, EPR, CB))

    wa = jnp.stack([W0_0, W1_0, W2_0])
    ba = jnp.stack([b0_0, b1_0, b2_0])
    wb = jnp.stack([W0_1, W1_1, W2_1])
    bb = jnp.stack([b0_1, b1_1, b2_1])
    wo = Wout.reshape(G, D, D)

    degp = _sc_deg(pk_rows).reshape(NC, G, NP)
    y0, y1, y2, w2, cc = _tc_mm(x_pad, wa, wb, wo, bb, bout)
    v0, v1, v2, dinv = _tc_scale(y0, y1, y2, degp)
    s0, s1, s2 = (a.reshape(NC, NP, D) for a in _sc_prop(v0, v1, v2, pk_rows))
    w0, w1, w2o = _tc_mid(s0, s1, s2, v0, v1, v2, dinv, ba, w2)
    t0, t1, t2 = (a.reshape(NC, NP, D) for a in _sc_prop(w0, w1, w2o, pk_rows))
    out = _tc_final(t0, t1, t2, w0, w1, w2o, dinv, cc)
    return out[:N]
